# Initial kernel scaffold; baseline (speedup 1.0000x reference)
import jax
import jax.numpy as jnp
from jax.experimental import pallas as pl


def kernel(position_ids, pos_embed):
    raise NotImplementedError("write your pallas kernel here")


# speedup vs baseline: 1.9166x; 1.9166x over previous
import functools

import jax
import jax.numpy as jnp
from jax import lax
from jax.experimental import pallas as pl
from jax.experimental.pallas import tpu as pltpu
from jax.experimental.pallas import tpu_sc as plsc


def _make_gather(V, D, B):
    info = plsc.get_sparse_core_info()
    NC, NS = info.num_cores, info.num_subcores
    NW = NC * NS
    assert B % NW == 0
    b_per_w = B // NW
    C = 64
    n_chunks = b_per_w // C
    assert b_per_w % C == 0

    mesh = plsc.VectorSubcoreMesh(core_axis_name="c", subcore_axis_name="s")

    @functools.partial(
        pl.kernel,
        mesh=mesh,
        out_type=jax.ShapeDtypeStruct((B, D), jnp.float32),
        scratch_types=[
            pltpu.VMEM((b_per_w,), jnp.int32),
            pltpu.VMEM((C, D), jnp.float32),
            pltpu.SemaphoreType.DMA,
        ],
    )
    def gather_kernel(idx_hbm, table_hbm, out_hbm, idx_v, rows_v, sem):
        wid = lax.axis_index("s") * NC + lax.axis_index("c")
        base = wid * b_per_w
        pltpu.sync_copy(idx_hbm.at[pl.ds(base, b_per_w)], idx_v)

        def body(g, carry):
            off = g * C
            pltpu.async_copy(
                table_hbm.at[idx_v.at[pl.ds(off, C)]], rows_v, sem
            ).wait()
            pltpu.sync_copy(rows_v, out_hbm.at[pl.ds(base + off, C)])
            return carry

        lax.fori_loop(0, n_chunks, body, 0)

    return gather_kernel


def kernel(position_ids, pos_embed):
    b, s = position_ids.shape
    v, d = pos_embed.shape
    flat_idx = position_ids.reshape(-1)
    out = _make_gather(v, d, flat_idx.shape[0])(flat_idx, pos_embed)
    return out.reshape(b, s, d)


# speedup vs baseline: 1.9249x; 1.0043x over previous
import functools

import jax
import jax.numpy as jnp
from jax import lax
from jax.experimental import pallas as pl
from jax.experimental.pallas import tpu as pltpu
from jax.experimental.pallas import tpu_sc as plsc


def _make_gather(V, D, B):
    info = plsc.get_sparse_core_info()
    NC, NS = info.num_cores, info.num_subcores
    NW = NC * NS
    assert B % NW == 0
    b_per_w = B // NW
    C = 32
    NBUF = 2
    n_chunks = b_per_w // C
    n_rounds = n_chunks // NBUF
    assert b_per_w % (C * NBUF) == 0

    mesh = plsc.VectorSubcoreMesh(core_axis_name="c", subcore_axis_name="s")

    @functools.partial(
        pl.kernel,
        mesh=mesh,
        out_type=jax.ShapeDtypeStruct((B, D), jnp.float32),
        scratch_types=[
            pltpu.VMEM((b_per_w,), jnp.int32),
        ]
        + [pltpu.VMEM((C, D), jnp.float32) for _ in range(NBUF)]
        + [pltpu.SemaphoreType.DMA for _ in range(2 * NBUF)],
    )
    def gather_kernel(idx_hbm, table_hbm, out_hbm, idx_v, *rest):
        bufs = rest[:NBUF]
        gsems = rest[NBUF : 2 * NBUF]
        ssems = rest[2 * NBUF :]
        wid = lax.axis_index("s") * NC + lax.axis_index("c")
        base = wid * b_per_w
        pltpu.sync_copy(idx_hbm.at[pl.ds(base, b_per_w)], idx_v)

        def start_gather(g, b):
            pltpu.async_copy(
                table_hbm.at[idx_v.at[pl.ds(g * C, C)]], bufs[b], gsems[b]
            )

        def wait_gather(b):
            pltpu.make_async_copy(
                table_hbm.at[idx_v.at[pl.ds(0, C)]], bufs[b], gsems[b]
            ).wait()

        def start_scatter(g, b):
            pltpu.async_copy(bufs[b], out_hbm.at[pl.ds(base + g * C, C)], ssems[b])

        def wait_scatter(b):
            pltpu.make_async_copy(
                bufs[b], out_hbm.at[pl.ds(base, C)], ssems[b]
            ).wait()

        for b in range(NBUF):
            start_gather(b, b)

        def body(s, carry):
            g0 = s * NBUF
            for b in range(NBUF):
                wait_gather(b)
                start_scatter(g0 + b, b)
            for b in range(NBUF):
                wait_scatter(b)

                @pl.when(g0 + b + NBUF < n_chunks)
                def _():
                    start_gather(g0 + b + NBUF, b)

            return carry

        lax.fori_loop(0, n_rounds, body, 0)

    return gather_kernel


def kernel(position_ids, pos_embed):
    b, s = position_ids.shape
    v, d = pos_embed.shape
    flat_idx = position_ids.reshape(-1)
    out = _make_gather(v, d, flat_idx.shape[0])(flat_idx, pos_embed)
    return out.reshape(b, s, d)


# speedup vs baseline: 1.9395x; 1.0076x over previous
import functools

import jax
import jax.numpy as jnp
from jax import lax
from jax.experimental import pallas as pl
from jax.experimental.pallas import tpu as pltpu
from jax.experimental.pallas import tpu_sc as plsc


def _make_gather(V, D, B):
    info = plsc.get_sparse_core_info()
    NC, NS = info.num_cores, info.num_subcores
    NW = NC * NS
    assert B % NW == 0
    b_per_w = B // NW
    C = 32
    NBUF = 2
    n_chunks = b_per_w // C
    n_rounds = n_chunks // NBUF
    assert b_per_w % (C * NBUF) == 0

    mesh = plsc.VectorSubcoreMesh(core_axis_name="c", subcore_axis_name="s")

    @functools.partial(
        pl.kernel,
        mesh=mesh,
        out_type=jax.ShapeDtypeStruct((B, D), jnp.float32),
        scratch_types=[
            pltpu.VMEM((b_per_w,), jnp.int32),
        ]
        + [pltpu.VMEM((C, D), jnp.float32) for _ in range(NBUF)]
        + [pltpu.SemaphoreType.DMA for _ in range(2 * NBUF)],
    )
    def gather_kernel(idx_hbm, table_hbm, out_hbm, idx_v, *rest):
        bufs = rest[:NBUF]
        gsems = rest[NBUF : 2 * NBUF]
        ssems = rest[2 * NBUF :]
        wid = lax.axis_index("s") * NC + lax.axis_index("c")
        base = wid * b_per_w
        pltpu.sync_copy(idx_hbm.at[pl.ds(base, b_per_w)], idx_v)

        def start_gather(g, b):
            pltpu.async_copy(
                table_hbm.at[idx_v.at[pl.ds(g * C, C)]], bufs[b], gsems[b]
            )

        def wait_gather(b):
            pltpu.make_async_copy(
                table_hbm.at[idx_v.at[pl.ds(0, C)]], bufs[b], gsems[b]
            ).wait()

        def start_scatter(g, b):
            pltpu.async_copy(bufs[b], out_hbm.at[pl.ds(base + g * C, C)], ssems[b])

        def wait_scatter(b):
            pltpu.make_async_copy(
                bufs[b], out_hbm.at[pl.ds(base, C)], ssems[b]
            ).wait()

        start_gather(0, 0)

        def body(s, carry):
            for b in range(NBUF):
                g = s * NBUF + b
                nb = 1 - b
                wait_gather(b)

                @pl.when(g >= 1)
                def _():
                    wait_scatter(nb)

                @pl.when(g + 1 < n_chunks)
                def _():
                    start_gather(g + 1, nb)

                start_scatter(g, b)
            return carry

        lax.fori_loop(0, n_rounds, body, 0)
        wait_scatter((n_chunks - 1) % NBUF)

    return gather_kernel


def kernel(position_ids, pos_embed):
    b, s = position_ids.shape
    v, d = pos_embed.shape
    flat_idx = position_ids.reshape(-1)
    out = _make_gather(v, d, flat_idx.shape[0])(flat_idx, pos_embed)
    return out.reshape(b, s, d)
